# Initial kernel scaffold; baseline (speedup 1.0000x reference)
import jax
import jax.numpy as jnp
from jax.experimental import pallas as pl


def kernel(x, edge_index, W1_l, b1, W1_r, W2_l, b2, W2_r):
    raise NotImplementedError("write your pallas kernel here")


# speedup vs baseline: 2.9636x; 2.9636x over previous
import functools

import jax
import jax.numpy as jnp
from jax import lax
from jax.experimental import pallas as pl
from jax.experimental.pallas import tpu as pltpu
from jax.experimental.pallas import tpu_sc as plsc

_N = 10000
_D = 128
_E = 320000
_NC = 2
_NS = 16
_L = 16
_NW = _NC * _NS
_CH = 128
_CPW = 80
_EPW = _CPW * _CH
_EPAD = _NW * _EPW
_RPT = 640
_NPAD = _NS * _RPT


def _make_agg(with_counts):
  mesh = plsc.VectorSubcoreMesh(
      core_axis_name="c", subcore_axis_name="s",
      num_cores=_NC, num_subcores=_NS)
  out_type = [jax.ShapeDtypeStruct((_NC, _NPAD, _D), jnp.float32)]
  scratch = [
      pltpu.VMEM((_CH,), jnp.int32),
      pltpu.VMEM((_CH,), jnp.int32),
      pltpu.VMEM((_CH, _D), jnp.float32),
      pltpu.VMEM_SHARED((_NPAD, _D), jnp.float32),
      pltpu.SemaphoreType.DMA,
  ]
  if with_counts:
    out_type.append(jax.ShapeDtypeStruct((_NC, _NPAD), jnp.float32))
    scratch.append(pltpu.VMEM((_CH,), jnp.float32))
    scratch.append(pltpu.VMEM_SHARED((_NPAD,), jnp.float32))

  def body(x_hbm, src_hbm, dst_hbm, zeros_hbm, *rest):
    if with_counts:
      part_out, cnt_out, src_v, dst_v, rows_v, acc_sh, sem, ones_v, cnt_sh = rest
    else:
      part_out, src_v, dst_v, rows_v, acc_sh, sem = rest
    c = lax.axis_index("c")
    s = lax.axis_index("s")
    wid = c * _NS + s
    row0 = s * _RPT
    pltpu.sync_copy(zeros_hbm.at[pl.ds(row0, _RPT)],
                    acc_sh.at[pl.ds(row0, _RPT)])
    if with_counts:
      for j in range(_CH // _L):
        ones_v[pl.ds(j * _L, _L)] = jnp.full((_L,), 1.0, jnp.float32)
      pltpu.sync_copy(zeros_hbm.at[s, pl.ds(0, _RPT)],
                      cnt_sh.at[pl.ds(row0, _RPT)])
    plsc.subcore_barrier()

    base_e = wid * _EPW

    def chunk(i, carry):
      base = base_e + i * _CH
      pltpu.sync_copy(src_hbm.at[pl.ds(base, _CH)], src_v)
      pltpu.sync_copy(dst_hbm.at[pl.ds(base, _CH)], dst_v)
      pltpu.async_copy(x_hbm.at[src_v], rows_v, sem).wait()
      pltpu.sync_copy(rows_v, acc_sh.at[dst_v], add=True)
      if with_counts:
        pltpu.sync_copy(ones_v, cnt_sh.at[dst_v], add=True)
      return carry
    lax.fori_loop(0, _CPW, chunk, 0)
    plsc.subcore_barrier()

    pltpu.sync_copy(acc_sh.at[pl.ds(row0, _RPT)],
                    part_out.at[c, pl.ds(row0, _RPT)])
    if with_counts:
      pltpu.sync_copy(cnt_sh.at[pl.ds(row0, _RPT)],
                      cnt_out.at[c, pl.ds(row0, _RPT)])

  return pl.kernel(body, out_type=out_type, mesh=mesh,
                   scratch_types=scratch)


_agg_counts = _make_agg(True)
_agg_only = _make_agg(False)


def _make_dense(relu):
  blk = 1280
  grid = (_NPAD // blk,)

  def body(p_ref, c_ref, x_ref, w_ref, b_ref, o_ref):
    p = p_ref[0] + p_ref[1]
    cnt = jnp.sum(c_ref[...], axis=0)
    inv = 1.0 / jnp.maximum(cnt, 1.0)
    mean = p * inv[:, None]
    acts = jnp.concatenate([mean, x_ref[...]], axis=1)
    h = jnp.dot(acts, w_ref[...], preferred_element_type=jnp.float32)
    h = h + b_ref[...]
    if relu:
      h = jnp.maximum(h, 0.0)
    o_ref[...] = h

  return pl.pallas_call(
      body,
      grid=grid,
      in_specs=[
          pl.BlockSpec((_NC, blk, _D), lambda i: (0, i, 0)),
          pl.BlockSpec((_NC, blk), lambda i: (0, i)),
          pl.BlockSpec((blk, _D), lambda i: (i, 0)),
          pl.BlockSpec((2 * _D, _D), lambda i: (0, 0)),
          pl.BlockSpec((1, _D), lambda i: (0, 0)),
      ],
      out_specs=pl.BlockSpec((blk, _D), lambda i: (i, 0)),
      out_shape=jax.ShapeDtypeStruct((_NPAD, _D), jnp.float32),
  )


_dense_relu = _make_dense(True)
_dense_lin = _make_dense(False)


def kernel(x, edge_index, W1_l, b1, W1_r, W2_l, b2, W2_r):
  src = edge_index[0]
  dst = edge_index[1]
  pad_idx = jnp.full((_EPAD - _E,), _N, jnp.int32)
  src_p = jnp.concatenate([src, pad_idx])
  dst_p = jnp.concatenate([dst, pad_idx])
  x_pad = jnp.zeros((_NPAD, _D), jnp.float32).at[:_N].set(x)
  zeros = jnp.zeros((_NPAD, _D), jnp.float32)
  W1 = jnp.concatenate([W1_l, W1_r], axis=0)
  W2 = jnp.concatenate([W2_l, W2_r], axis=0)

  parts1, cnts = _agg_counts(x_pad, src_p, dst_p, zeros)
  h = _dense_relu(parts1, cnts, x_pad, W1, b1.reshape(1, _D))
  (parts2,) = _agg_only(h, src_p, dst_p, zeros)
  out = _dense_lin(parts2, cnts, h, W2, b2.reshape(1, _D))
  return out[:_N]


# speedup vs baseline: 5.9897x; 2.0211x over previous
import jax
import jax.numpy as jnp
from jax import lax
from jax.experimental import pallas as pl
from jax.experimental.pallas import tpu as pltpu
from jax.experimental.pallas import tpu_sc as plsc

_N = 10000
_D = 128
_E = 320000
_NC = 2
_NS = 16
_L = 16
_HD = _D // _NC
_CH = 128
_CPT = 160
_EPT = _CPT * _CH
_EPAD = _NS * _EPT
_RPT = 640
_NPAD = _NS * _RPT
_NBUF = 4


def _make_agg(with_counts):
  mesh = plsc.VectorSubcoreMesh(
      core_axis_name="c", subcore_axis_name="s",
      num_cores=_NC, num_subcores=_NS)
  out_type = [jax.ShapeDtypeStruct((_NC, _NPAD, _HD), jnp.float32)]
  scratch = [
      pltpu.VMEM((_CPT, _CH), jnp.int32),
      pltpu.VMEM((_CPT, _CH), jnp.int32),
      pltpu.VMEM((_NBUF, _CH, _HD), jnp.float32),
      pltpu.VMEM_SHARED((_NPAD, _HD), jnp.float32),
      [pltpu.SemaphoreType.DMA] * _NBUF,
  ]
  if with_counts:
    out_type.append(jax.ShapeDtypeStruct((1, _NPAD), jnp.float32))
    scratch.append(pltpu.VMEM((_CH,), jnp.float32))
    scratch.append(pltpu.VMEM_SHARED((_NPAD,), jnp.float32))

  def body(x_hbm, src_hbm, dst_hbm, z2_hbm, z1_hbm, *rest):
    if with_counts:
      part_out, cnt_out, src_v, dst_v, rows_v, acc_sh, sems, ones_v, cnt_sh = rest
    else:
      part_out, src_v, dst_v, rows_v, acc_sh, sems = rest
    c = lax.axis_index("c")
    s = lax.axis_index("s")
    row0 = s * _RPT
    pltpu.sync_copy(src_hbm.at[s], src_v)
    pltpu.sync_copy(dst_hbm.at[s], dst_v)
    pltpu.sync_copy(z2_hbm.at[pl.ds(row0, _RPT)],
                    acc_sh.at[pl.ds(row0, _RPT)])
    if with_counts:
      for j in range(_CH // _L):
        ones_v[pl.ds(j * _L, _L)] = jnp.full((_L,), 1.0, jnp.float32)
      pltpu.sync_copy(z1_hbm.at[pl.ds(row0, _RPT)],
                      cnt_sh.at[pl.ds(row0, _RPT)])
    plsc.subcore_barrier()

    xc = x_hbm.at[c]

    for b in range(_NBUF):
      pltpu.async_copy(xc.at[src_v.at[b]], rows_v.at[b], sems[b])

    def block(i, carry):
      for b in range(_NBUF):
        g = i * _NBUF + b
        pltpu.make_async_copy(xc.at[src_v.at[g]], rows_v.at[b],
                              sems[b]).wait()
        pltpu.sync_copy(rows_v.at[b], acc_sh.at[dst_v.at[g]], add=True)
        if with_counts:
          @pl.when(c == 0)
          def _():
            pltpu.sync_copy(ones_v, cnt_sh.at[dst_v.at[g]], add=True)
        @pl.when(g + _NBUF < _CPT)
        def _():
          pltpu.async_copy(xc.at[src_v.at[g + _NBUF]], rows_v.at[b],
                           sems[b])
      return carry
    lax.fori_loop(0, _CPT // _NBUF, block, 0)
    plsc.subcore_barrier()

    pltpu.sync_copy(acc_sh.at[pl.ds(row0, _RPT)],
                    part_out.at[c, pl.ds(row0, _RPT)])
    if with_counts:
      @pl.when(c == 0)
      def _():
        pltpu.sync_copy(cnt_sh.at[pl.ds(row0, _RPT)],
                        cnt_out.at[0, pl.ds(row0, _RPT)])

  return pl.kernel(body, out_type=out_type, mesh=mesh,
                   scratch_types=scratch,
                   compiler_params=pltpu.CompilerParams(
                       use_tc_tiling_on_sc=False))


_agg_counts = _make_agg(True)
_agg_only = _make_agg(False)


def _make_dense(relu, split_out):
  blk = 1280
  grid = (_NPAD // blk,)

  def body(p_ref, c_ref, x_ref, w_ref, b_ref, o_ref):
    p = jnp.concatenate([p_ref[0], p_ref[1]], axis=1)
    xin = jnp.concatenate([x_ref[0], x_ref[1]], axis=1)
    cnt = c_ref[0]
    inv = 1.0 / jnp.maximum(cnt, 1.0)
    mean = p * inv[:, None]
    acts = jnp.concatenate([mean, xin], axis=1)
    h = jnp.dot(acts, w_ref[...], preferred_element_type=jnp.float32)
    h = h + b_ref[...]
    if relu:
      h = jnp.maximum(h, 0.0)
    if split_out:
      o_ref[0] = h[:, :_HD]
      o_ref[1] = h[:, _HD:]
    else:
      o_ref[...] = h

  if split_out:
    out_shape = jax.ShapeDtypeStruct((_NC, _NPAD, _HD), jnp.float32)
    out_spec = pl.BlockSpec((_NC, blk, _HD), lambda i: (0, i, 0))
  else:
    out_shape = jax.ShapeDtypeStruct((_NPAD, _D), jnp.float32)
    out_spec = pl.BlockSpec((blk, _D), lambda i: (i, 0))

  return pl.pallas_call(
      body,
      grid=grid,
      in_specs=[
          pl.BlockSpec((_NC, blk, _HD), lambda i: (0, i, 0)),
          pl.BlockSpec((1, blk), lambda i: (0, i)),
          pl.BlockSpec((_NC, blk, _HD), lambda i: (0, i, 0)),
          pl.BlockSpec((2 * _D, _D), lambda i: (0, 0)),
          pl.BlockSpec((1, _D), lambda i: (0, 0)),
      ],
      out_specs=out_spec,
      out_shape=out_shape,
  )


_dense_relu = _make_dense(True, True)
_dense_lin = _make_dense(False, False)


def kernel(x, edge_index, W1_l, b1, W1_r, W2_l, b2, W2_r):
  src = edge_index[0]
  dst = edge_index[1]
  pad_idx = jnp.full((_EPAD - _E,), _N, jnp.int32)
  src_p = jnp.concatenate([src, pad_idx]).reshape(_NS, _CPT, _CH)
  dst_p = jnp.concatenate([dst, pad_idx]).reshape(_NS, _CPT, _CH)
  x_pad = jnp.zeros((_NPAD, _D), jnp.float32).at[:_N].set(x)
  x_split = x_pad.reshape(_NPAD, _NC, _HD).transpose(1, 0, 2)
  z2 = jnp.zeros((_NPAD, _HD), jnp.float32)
  z1 = jnp.zeros((_NPAD,), jnp.float32)
  W1 = jnp.concatenate([W1_l, W1_r], axis=0)
  W2 = jnp.concatenate([W2_l, W2_r], axis=0)

  parts1, cnts = _agg_counts(x_split, src_p, dst_p, z2, z1)
  h_split = _dense_relu(parts1, cnts, x_split, W1, b1.reshape(1, _D))
  (parts2,) = _agg_only(h_split, src_p, dst_p, z2, z1)
  out = _dense_lin(parts2, cnts, h_split, W2, b2.reshape(1, _D))
  return out[:_N]
